# baseline (speedup 1.0000x reference)
import functools

import jax
import jax.numpy as jnp
from jax.experimental import pallas as pl

T = 2048
D = 1024
E = 16
TOPK = 8
NG = 4
TG = 2
FF = 512
SFF = 1024
SCALE = 2.5
EPP = 2
NRS = E // EPP


def _routing_kernel(hs_ref, gw_ref, bias_ref, comb_ref):
    logits_t = jax.lax.dot_general(
        gw_ref[...], hs_ref[...],
        dimension_numbers=(((1,), (1,)), ((), ())),
        preferred_element_type=jnp.float32,
    )
    scores_t = jax.nn.sigmoid(logits_t)
    swb = scores_t + bias_ref[...]

    gs_rows = []
    for g in range(NG):
        r = [swb[4 * g + i:4 * g + i + 1, :] for i in range(4)]
        ps = [r[i] + r[j] for i in range(4) for j in range(i + 1, 4)]
        acc = ps[0]
        for p in ps[1:]:
            acc = jnp.maximum(acc, p)
        gs_rows.append(acc)
    gs = jnp.concatenate(gs_rows, axis=0)

    grank = jnp.zeros_like(gs)
    grow_ids = jax.lax.broadcasted_iota(jnp.int32, gs.shape, 0)
    for j in range(NG):
        xj = gs[j:j + 1, :]
        gt = (xj > gs).astype(jnp.float32)
        ge = (xj >= gs).astype(jnp.float32)
        beats = jnp.where(grow_ids > j, ge, gt)
        beats = jnp.where(grow_ids == j, 0.0, beats)
        grank = grank + beats
    gmask = (grank < TG).astype(jnp.float32)

    score_mask = jnp.concatenate(
        [gmask[g:g + 1, :] for g in range(NG) for _ in range(4)], axis=0
    )
    swb_m = swb * score_mask

    rank = jnp.zeros_like(swb_m)
    row_ids = jax.lax.broadcasted_iota(jnp.int32, swb_m.shape, 0)
    for j in range(E):
        xj = swb_m[j:j + 1, :]
        gt = (xj > swb_m).astype(jnp.float32)
        ge = (xj >= swb_m).astype(jnp.float32)
        beats = jnp.where(row_ids > j, ge, gt)
        beats = jnp.where(row_ids == j, 0.0, beats)
        rank = rank + beats
    kmask = (rank < TOPK).astype(jnp.float32)

    sc = scores_t * kmask
    denom = jnp.sum(sc, axis=0, keepdims=True) + 1e-20
    comb_ref[...] = sc / denom * SCALE


NS = E + 2


def _expert_kernel(hs_ref, wg_ref, wu_ref, wd_ref,
                   sg_ref, su_ref, sd_ref, comb_ref, out_ref):
    e = pl.program_id(0)
    bf = jnp.bfloat16
    h = hs_ref[...]

    def mlp(wg, wu, wd, c):
        a = jnp.dot(h, wg.astype(bf), preferred_element_type=jnp.float32)
        u = jnp.dot(h, wu.astype(bf), preferred_element_type=jnp.float32)
        act = (a * jax.nn.sigmoid(a)) * u
        if c is not None:
            act = act * c
        return jnp.dot(act.astype(bf), wd.astype(bf),
                       preferred_element_type=jnp.float32)

    @pl.when(e == 0)
    def _init():
        out_ref[...] = mlp(wg_ref[0], wu_ref[0], wd_ref[0], comb_ref[0])

    @pl.when((e > 0) & (e < E))
    def _routed():
        out_ref[...] += mlp(wg_ref[0], wu_ref[0], wd_ref[0], comb_ref[0])

    @pl.when(e >= E)
    def _shared():
        out_ref[...] += mlp(sg_ref[0], su_ref[0], sd_ref[0], None)


@functools.partial(jax.jit, static_argnames=())
def kernel(hidden_states, gate_weight, e_score_correction_bias,
           w_gate, w_up, w_down, s_gate, s_up, s_down):
    hs32 = hidden_states.astype(jnp.float32)
    comb = pl.pallas_call(
        _routing_kernel,
        out_shape=jax.ShapeDtypeStruct((E, T), jnp.float32),
    )(hs32, gate_weight.astype(jnp.float32),
      e_score_correction_bias.astype(jnp.float32).reshape(E, 1))

    hsb = hidden_states.astype(jnp.bfloat16)
    comb_r = comb[:, :, None]

    sg2 = s_gate.reshape(D, 2, FF).transpose(1, 0, 2)
    su2 = s_up.reshape(D, 2, FF).transpose(1, 0, 2)
    sd2 = s_down.reshape(2, FF, D)

    BT = T // 2
    rid = lambda e, t: (jnp.minimum(e, E - 1), 0, 0)
    sid = lambda e, t: (jnp.clip(e - E, 0, 1), 0, 0)
    cid = lambda e, t: (jnp.minimum(e, E - 1), t, 0)
    out = pl.pallas_call(
        _expert_kernel,
        grid=(NS, 2),
        in_specs=[
            pl.BlockSpec((BT, D), lambda e, t: (t, 0)),
            pl.BlockSpec((1, D, FF), rid),
            pl.BlockSpec((1, D, FF), rid),
            pl.BlockSpec((1, FF, D), rid),
            pl.BlockSpec((1, D, FF), sid),
            pl.BlockSpec((1, D, FF), sid),
            pl.BlockSpec((1, FF, D), sid),
            pl.BlockSpec((1, BT, 1), cid),
        ],
        out_specs=pl.BlockSpec((BT, D), lambda e, t: (t, 0)),
        out_shape=jax.ShapeDtypeStruct((T, D), jnp.float32),
    )(hsb, w_gate, w_up, w_down, sg2, su2, sd2, comb_r)
    return out


# speedup vs baseline: 1.1034x; 1.1034x over previous
import functools

import jax
import jax.numpy as jnp
from jax.experimental import pallas as pl

T = 2048
D = 1024
E = 16
TOPK = 8
NG = 4
TG = 2
FF = 512
SFF = 1024
SCALE = 2.5
EPP = 2
NRS = E // EPP


def _routing_kernel(hs_ref, gw_ref, bias_ref, comb_ref):
    logits_t = jax.lax.dot_general(
        gw_ref[...], hs_ref[...],
        dimension_numbers=(((1,), (1,)), ((), ())),
        preferred_element_type=jnp.float32,
    )
    scores_t = jax.nn.sigmoid(logits_t)
    swb = scores_t + bias_ref[...]

    gs_rows = []
    for g in range(NG):
        r = [swb[4 * g + i:4 * g + i + 1, :] for i in range(4)]
        ps = [r[i] + r[j] for i in range(4) for j in range(i + 1, 4)]
        acc = ps[0]
        for p in ps[1:]:
            acc = jnp.maximum(acc, p)
        gs_rows.append(acc)
    gs = jnp.concatenate(gs_rows, axis=0)

    grank = jnp.zeros_like(gs)
    grow_ids = jax.lax.broadcasted_iota(jnp.int32, gs.shape, 0)
    for j in range(NG):
        xj = gs[j:j + 1, :]
        gt = (xj > gs).astype(jnp.float32)
        ge = (xj >= gs).astype(jnp.float32)
        beats = jnp.where(grow_ids > j, ge, gt)
        beats = jnp.where(grow_ids == j, 0.0, beats)
        grank = grank + beats
    gmask = (grank < TG).astype(jnp.float32)

    score_mask = jnp.concatenate(
        [gmask[g:g + 1, :] for g in range(NG) for _ in range(4)], axis=0
    )
    swb_m = swb * score_mask

    rank = jnp.zeros_like(swb_m)
    row_ids = jax.lax.broadcasted_iota(jnp.int32, swb_m.shape, 0)
    for j in range(E):
        xj = swb_m[j:j + 1, :]
        gt = (xj > swb_m).astype(jnp.float32)
        ge = (xj >= swb_m).astype(jnp.float32)
        beats = jnp.where(row_ids > j, ge, gt)
        beats = jnp.where(row_ids == j, 0.0, beats)
        rank = rank + beats
    kmask = (rank < TOPK).astype(jnp.float32)

    sc = scores_t * kmask
    denom = jnp.sum(sc, axis=0, keepdims=True) + 1e-20
    comb_ref[...] = sc / denom * SCALE


NS = E + 2


def _expert_kernel(hs_ref, wg_ref, wu_ref, wd_ref,
                   sg_ref, su_ref, sd_ref, comb_ref, out_ref):
    e = pl.program_id(0)
    bf = jnp.bfloat16
    h = hs_ref[...]

    def mlp(wg, wu, wd, c):
        a = jnp.dot(h, wg.astype(bf), preferred_element_type=jnp.float32)
        u = jnp.dot(h, wu.astype(bf), preferred_element_type=jnp.float32)
        act = (a * jax.nn.sigmoid(a)) * u
        if c is not None:
            act = act * c
        return jnp.dot(act.astype(bf), wd.astype(bf),
                       preferred_element_type=jnp.float32)

    @pl.when(e == 0)
    def _init():
        out_ref[...] = mlp(wg_ref[0], wu_ref[0], wd_ref[0], comb_ref[0])

    @pl.when((e > 0) & (e < E))
    def _routed():
        out_ref[...] += mlp(wg_ref[0], wu_ref[0], wd_ref[0], comb_ref[0])

    @pl.when(e >= E)
    def _shared():
        out_ref[...] += mlp(sg_ref[0], su_ref[0], sd_ref[0], None)


@functools.partial(jax.jit, static_argnames=())
def kernel(hidden_states, gate_weight, e_score_correction_bias,
           w_gate, w_up, w_down, s_gate, s_up, s_down):
    hs32 = hidden_states.astype(jnp.float32)
    comb = pl.pallas_call(
        _routing_kernel,
        out_shape=jax.ShapeDtypeStruct((E, T), jnp.float32),
    )(hs32, gate_weight.astype(jnp.float32),
      e_score_correction_bias.astype(jnp.float32).reshape(E, 1))

    hsb = hidden_states.astype(jnp.bfloat16)
    comb_r = comb[:, :, None]

    sg2 = s_gate.reshape(D, 2, FF).transpose(1, 0, 2)
    su2 = s_up.reshape(D, 2, FF).transpose(1, 0, 2)
    sd2 = s_down.reshape(2, FF, D)

    rid = lambda e: (jnp.minimum(e, E - 1), 0, 0)
    sid = lambda e: (jnp.clip(e - E, 0, 1), 0, 0)
    out = pl.pallas_call(
        _expert_kernel,
        grid=(NS,),
        in_specs=[
            pl.BlockSpec((T, D), lambda e: (0, 0)),
            pl.BlockSpec((1, D, FF), rid),
            pl.BlockSpec((1, D, FF), rid),
            pl.BlockSpec((1, FF, D), rid),
            pl.BlockSpec((1, D, FF), sid),
            pl.BlockSpec((1, D, FF), sid),
            pl.BlockSpec((1, FF, D), sid),
            pl.BlockSpec((1, T, 1), rid),
        ],
        out_specs=pl.BlockSpec((T, D), lambda e: (0, 0)),
        out_shape=jax.ShapeDtypeStruct((T, D), jnp.float32),
    )(hsb, w_gate, w_up, w_down, sg2, su2, sd2, comb_r)
    return out
